# baseline (speedup 1.0000x reference)
import functools

import jax
import jax.numpy as jnp
from jax import lax
from jax.experimental import pallas as pl
from jax.experimental.pallas import tpu as pltpu
from jax.experimental.pallas import tpu_sc as plsc

_N = 10000
_E = 320000
_G = 100

_NC = 2
_NS = 16
_NW = _NC * _NS
_NP = 10240
_CH = 80
_C = 125
_EP = _NW * _CH * _C

_R = 320
_PW = _NP // _NW
_PC = 64
_GP = 104
_HD = 240
_HL = _HD // 16

_sc_mesh = dict(core_axis_name="c", subcore_axis_name="s", num_cores=_NC,
                num_subcores=_NS)
_sc_params = pltpu.CompilerParams(use_tc_tiling_on_sc=False,
                                  needs_layout_passes=False)


def _make_sc_segsum(dp):
    dl = dp // 16
    zrows = 64
    rows_per_sub = _NP // _NS

    @functools.partial(
        pl.kernel,
        out_type=jax.ShapeDtypeStruct((_NC, _NP, dp), jnp.float32),
        mesh=plsc.VectorSubcoreMesh(**_sc_mesh),
        compiler_params=_sc_params,
        scratch_types=[
            pltpu.VMEM((_CH, _C), jnp.int32),
            pltpu.VMEM((_CH, _C), jnp.int32),
            pltpu.VMEM((_C, dp), jnp.float32),
            pltpu.VMEM((_C, dp), jnp.float32),
            pltpu.VMEM((zrows, dp), jnp.float32),
            pltpu.VMEM_SHARED((_NP, dp), jnp.float32),
            pltpu.SemaphoreType.DMA,
            pltpu.SemaphoreType.DMA,
        ],
    )
    def k(p_hbm, col_hbm, row_hbm, out_hbm, colv, rowv, buf0, buf1, zbuf, acc, sem0, sem1):
        cid = lax.axis_index("c")
        sid = lax.axis_index("s")
        wid = sid * _NC + cid

        def zb_body(i, carry):
            r = i // dl
            j = i - r * dl
            zbuf[r, pl.ds(j * 16, 16)] = jnp.zeros((16,), jnp.float32)
            return carry

        lax.fori_loop(0, zrows * dl, zb_body, 0)

        base = sid * rows_per_sub

        def zc_body(i, carry):
            pltpu.sync_copy(zbuf, acc.at[pl.ds(base + i * zrows, zrows)])
            return carry

        lax.fori_loop(0, rows_per_sub // zrows, zc_body, 0)

        pltpu.sync_copy(col_hbm.at[wid], colv)
        pltpu.sync_copy(row_hbm.at[wid], rowv)

        plsc.subcore_barrier()

        pltpu.async_copy(p_hbm.at[colv.at[0]], buf0, sem0)

        def ch_body(g, carry):
            k0 = 2 * g
            k1 = 2 * g + 1
            pltpu.make_async_copy(p_hbm.at[colv.at[k0]], buf0, sem0).wait()
            pltpu.async_copy(p_hbm.at[colv.at[k1]], buf1, sem1)
            pltpu.sync_copy(buf0, acc.at[rowv.at[k0]], add=True)
            pltpu.make_async_copy(p_hbm.at[colv.at[k1]], buf1, sem1).wait()

            @pl.when(g + 1 < _CH // 2)
            def _():
                pltpu.async_copy(p_hbm.at[colv.at[k1 + 1]], buf0, sem0)

            pltpu.sync_copy(buf1, acc.at[rowv.at[k1]], add=True)
            return carry

        lax.fori_loop(0, _CH // 2, ch_body, 0)

        plsc.subcore_barrier()

        pltpu.sync_copy(
            acc.at[pl.ds(base, rows_per_sub)],
            out_hbm.at[cid, pl.ds(base, rows_per_sub)],
        )

    return k


_sc_cache = {}


def _sc_segsum(dp):
    if dp not in _sc_cache:
        _sc_cache[dp] = _make_sc_segsum(dp)
    return _sc_cache[dp]


def _make_sc_pool():
    if "pool" in _sc_cache:
        return _sc_cache["pool"]
    k = functools.partial(
        pl.kernel,
        out_type=jax.ShapeDtypeStruct((_NW, 3, _GP, _HD), jnp.float32),
        mesh=plsc.VectorSubcoreMesh(**_sc_mesh),
        compiler_params=_sc_params,
        scratch_types=[
            pltpu.VMEM((_PC, _HD), jnp.float32),
            pltpu.VMEM((_PW,), jnp.int32),
            pltpu.VMEM((_GP, _HD), jnp.float32),
            pltpu.VMEM((_GP, _HD), jnp.float32),
            pltpu.VMEM((_GP, _HD), jnp.float32),
        ],
    )(_sc_pool_body)
    _sc_cache["pool"] = k
    return k


def _sc_pool_body(h_hbm, gid_hbm, out_hbm, chunk, gidv, accs, accn, accx):
    cid = lax.axis_index("c")
    sid = lax.axis_index("s")
    wid = sid * _NC + cid
    rbase = wid * _PW

    pltpu.sync_copy(gid_hbm.at[pl.ds(rbase, _PW)], gidv)

    def zc_body(i, carry):
        accs[i, pl.ds(_HD - 16, 16)] = jnp.zeros((16,), jnp.float32)
        return carry

    lax.fori_loop(0, _GP, zc_body, 0)

    zero = jnp.zeros((16,), jnp.float32)
    init = (jnp.int32(-1),) + tuple(zero for _ in range(3 * _HL))

    def chunk_body(c, carry):
        pltpu.sync_copy(h_hbm.at[pl.ds(rbase + c * _PC, _PC)], chunk)

        def vec_body(k, carry):
            gvec = gidv[pl.ds(c * _PC + k * 16, 16)]
            for l in range(16):
                prev = carry[0]
                ss = carry[1:1 + _HL]
                nn = carry[1 + _HL:1 + 2 * _HL]
                xx = carry[1 + 2 * _HL:]
                lane = lax.iota(jnp.int32, 16) == l
                g = jnp.max(jnp.where(lane, gvec, -1))
                is_new = g != prev

                @pl.when(jnp.logical_and(is_new, prev >= 0))
                def _():
                    for j in range(_HL):
                        accs[prev, pl.ds(j * 16, 16)] = ss[j]
                        accn[prev, pl.ds(j * 16, 16)] = nn[j]
                        accx[prev, pl.ds(j * 16, 16)] = xx[j]

                r = k * 16 + l
                vs = [chunk[r, pl.ds(j * 16, 16)] for j in range(_HL)]
                ss = tuple(jnp.where(is_new, vs[j], ss[j] + vs[j]) for j in range(_HL))
                nn = tuple(jnp.where(is_new, vs[j], jnp.minimum(nn[j], vs[j])) for j in range(_HL))
                xx = tuple(jnp.where(is_new, vs[j], jnp.maximum(xx[j], vs[j])) for j in range(_HL))
                carry = (g,) + ss + nn + xx
            return carry

        return lax.fori_loop(0, _PC // 16, vec_body, carry)

    fin = lax.fori_loop(0, _PW // _PC, chunk_body, init)
    prev = fin[0]
    for j in range(_HL):
        accs[prev, pl.ds(j * 16, 16)] = fin[1 + j]
        accn[prev, pl.ds(j * 16, 16)] = fin[1 + _HL + j]
        accx[prev, pl.ds(j * 16, 16)] = fin[1 + 2 * _HL + j]

    pltpu.sync_copy(accs, out_hbm.at[wid, 0])
    pltpu.sync_copy(accn, out_hbm.at[wid, 1])
    pltpu.sync_copy(accx, out_hbm.at[wid, 2])


def _proj_body(x_ref, nk_ref, sk_ref, pp_ref, sh_ref):
    xb = x_ref[...]
    ones = jnp.where(lax.broadcasted_iota(jnp.int32, (_R, 64), 1) == 50, 1.0, 0.0)
    pp_ref[...] = xb @ nk_ref[...] + ones
    sh_ref[...] = xb @ sk_ref[...]


def _combine_body(sp_ref, sh_ref, bn_ref, bs_ref, nkt_ref, nkb_ref, skt_ref,
                  skb_ref, pp_ref, shn_ref, *, u, uo, dpi, dpo):
    s = sp_ref[0] + sp_ref[1]
    deg = jnp.maximum(s[:, u:u + 1], 1.0)
    hm = jnp.maximum(s / deg + bn_ref[...], 0.0)
    hs = jnp.maximum(sh_ref[...] + bs_ref[...], 0.0)
    ones = jnp.where(lax.broadcasted_iota(jnp.int32, (_R, dpo), 1) == uo, 1.0, 0.0)
    pp_ref[...] = hm @ nkt_ref[...] + hs @ nkb_ref[...] + ones
    shn_ref[...] = hm @ skt_ref[...] + hs @ skb_ref[...]


def _final_body(sp_ref, sh_ref, bn_ref, bs_ref, out_ref):
    s = sp_ref[0] + sp_ref[1]
    deg = jnp.maximum(s[:, 100:101], 1.0)
    hm = jnp.maximum(s / deg + bn_ref[...], 0.0)
    hs = jnp.maximum(sh_ref[...] + bs_ref[...], 0.0)
    cnt = jnp.where(lax.broadcasted_iota(jnp.int32, (_R, 16), 1) == 0, 1.0, 0.0)
    out_ref[...] = jnp.concatenate([hm, hs, cnt], axis=1)


def _head_body(parts_ref, W1_ref, b1_ref, W2_ref, b2_ref, W3_ref, b3_ref, out_ref):
    p = parts_ref[...]
    valid = p[:, 0, :, _HD - 16:_HD - 15] > 0.0
    sums = jnp.sum(jnp.where(valid, p[:, 0], 0.0), axis=0)
    mins = jnp.min(jnp.where(valid, p[:, 1], jnp.inf), axis=0)
    maxs = jnp.max(jnp.where(valid, p[:, 2], -jnp.inf), axis=0)
    cnt = jnp.maximum(sums[:, _HD - 16:_HD - 15], 1.0)
    mean = sums / cnt
    pool = jnp.concatenate(
        [mean[:, 0:100], mean[:, 112:212], mins[:, 0:100], mins[:, 112:212],
         maxs[:, 0:100], maxs[:, 112:212], sums[:, 0:100], sums[:, 112:212]],
        axis=1,
    )
    out = pool @ W1_ref[...] + b1_ref[...]
    out = out @ W2_ref[...] + b2_ref[...]
    out = out @ W3_ref[...] + b3_ref[...]
    out_ref[...] = out[:_G]


def _row_specs(widths):
    return [pl.BlockSpec((_R, w), lambda i: (i, 0)) for w in widths]


def _full_specs(shapes):
    return [
        pl.BlockSpec(s, lambda i, _n=len(s): (0,) * _n) for s in shapes
    ]


def _pad2(a, shape, r0=0, c0=0):
    return jnp.zeros(shape, jnp.float32).at[r0:r0 + a.shape[0], c0:c0 + a.shape[1]].set(a)


def kernel(x, edge_index, node_graph_index, self_k0, neigh_k0, bias0, self_k1, neigh_k1, bias1, self_kF, neigh_kF, biasF, W1, b1, W2, b2, W3, b3):
    col3 = jnp.concatenate(
        [edge_index[1], jnp.zeros((_EP - _E,), jnp.int32)]).reshape(_NW, _CH, _C)
    pad_rows = _N + 16 + jnp.arange(_EP - _E, dtype=jnp.int32) % (_NP - _N - 16)
    row3 = jnp.concatenate([edge_index[0], pad_rows]).reshape(_NW, _CH, _C)
    xp = jnp.pad(x, ((0, _NP - _N), (0, 0)))
    gidp = jnp.concatenate([node_graph_index, jnp.full((_NP - _N,), _G, jnp.int32)])

    nk0 = _pad2(neigh_k0, (128, 64))
    sk0 = _pad2(self_k0, (128, 64))
    w1 = [_pad2(m[a:a + 50], (64, 80)) for m in (neigh_k1, self_k1) for a in (0, 50)]
    w2 = [_pad2(m[a:a + 75], (80, 112)) for m in (neigh_kF, self_kF) for a in (0, 75)]
    bn0, bs0 = _pad2(bias0[None, :50], (1, 64)), _pad2(bias0[None, 50:], (1, 64))
    bn1, bs1 = _pad2(bias1[None, :75], (1, 80)), _pad2(bias1[None, 75:], (1, 80))
    bn2, bs2 = _pad2(biasF[None, :100], (1, 112)), _pad2(biasF[None, 100:], (1, 112))

    grid = _NP // _R

    p0, sh0 = pl.pallas_call(
        _proj_body,
        grid=grid,
        in_specs=_row_specs([128]) + _full_specs([(128, 64), (128, 64)]),
        out_specs=_row_specs([64, 64]),
        out_shape=[jax.ShapeDtypeStruct((_NP, 64), jnp.float32)] * 2,
    )(xp, nk0, sk0)

    s0 = _sc_segsum(64)(p0, col3, row3)

    def combine(sp, sh, bn, bs, ws, u, uo, dpi, dpo):
        return pl.pallas_call(
            functools.partial(_combine_body, u=u, uo=uo, dpi=dpi, dpo=dpo),
            grid=grid,
            in_specs=[pl.BlockSpec((_NC, _R, dpi), lambda i: (0, i, 0))]
            + _row_specs([dpi])
            + _full_specs([(1, dpi), (1, dpi), (dpi, dpo), (dpi, dpo), (dpi, dpo), (dpi, dpo)]),
            out_specs=_row_specs([dpo, dpo]),
            out_shape=[jax.ShapeDtypeStruct((_NP, dpo), jnp.float32)] * 2,
        )(sp, sh, bn, bs, ws[0], ws[1], ws[2], ws[3])

    p1, sh1 = combine(s0, sh0, bn0, bs0, w1, 50, 75, 64, 80)
    s1 = _sc_segsum(80)(p1, col3, row3)
    p2, sh2 = combine(s1, sh1, bn1, bs1, w2, 75, 100, 80, 112)
    s2 = _sc_segsum(112)(p2, col3, row3)

    h3 = pl.pallas_call(
        _final_body,
        grid=grid,
        in_specs=[pl.BlockSpec((_NC, _R, 112), lambda i: (0, i, 0))]
        + _row_specs([112])
        + _full_specs([(1, 112), (1, 112)]),
        out_specs=pl.BlockSpec((_R, _HD), lambda i: (i, 0)),
        out_shape=jax.ShapeDtypeStruct((_NP, _HD), jnp.float32),
    )(s2, sh2, bn2, bs2)

    parts = _make_sc_pool()(h3, gidp)

    out = pl.pallas_call(
        _head_body,
        out_shape=jax.ShapeDtypeStruct((_G, 10), jnp.float32),
    )(parts, W1, b1[None], W2, b2[None], W3, b3[None])
    return out


# speedup vs baseline: 1.0963x; 1.0963x over previous
import functools

import jax
import jax.numpy as jnp
from jax import lax
from jax.experimental import pallas as pl
from jax.experimental.pallas import tpu as pltpu
from jax.experimental.pallas import tpu_sc as plsc

_N = 10000
_E = 320000
_G = 100

_NC = 2
_NS = 16
_NW = _NC * _NS
_NP = 10240
_CH = 80
_C = 125
_EP = _NW * _CH * _C

_R = 1280
_PW = _NP // _NW
_PC = 64
_GP = 104
_HD = 240
_HL = _HD // 16

_sc_mesh = dict(core_axis_name="c", subcore_axis_name="s", num_cores=_NC,
                num_subcores=_NS)
_sc_params = pltpu.CompilerParams(use_tc_tiling_on_sc=False,
                                  needs_layout_passes=False)


def _make_sc_segsum(dp):
    dl = dp // 16
    zrows = 64
    rows_per_sub = _NP // _NS

    @functools.partial(
        pl.kernel,
        out_type=jax.ShapeDtypeStruct((_NC, _NP, dp), jnp.float32),
        mesh=plsc.VectorSubcoreMesh(**_sc_mesh),
        compiler_params=_sc_params,
        scratch_types=[
            pltpu.VMEM((_CH, _C), jnp.int32),
            pltpu.VMEM((_CH, _C), jnp.int32),
            pltpu.VMEM((_C, dp), jnp.float32),
            pltpu.VMEM((_C, dp), jnp.float32),
            pltpu.VMEM((zrows, dp), jnp.float32),
            pltpu.VMEM_SHARED((_NP, dp), jnp.float32),
            pltpu.SemaphoreType.DMA,
            pltpu.SemaphoreType.DMA,
        ],
    )
    def k(p_hbm, col_hbm, row_hbm, out_hbm, colv, rowv, buf0, buf1, zbuf, acc, sem0, sem1):
        cid = lax.axis_index("c")
        sid = lax.axis_index("s")
        wid = sid * _NC + cid

        def zb_body(i, carry):
            r = i // dl
            j = i - r * dl
            zbuf[r, pl.ds(j * 16, 16)] = jnp.zeros((16,), jnp.float32)
            return carry

        lax.fori_loop(0, zrows * dl, zb_body, 0)

        base = sid * rows_per_sub

        def zc_body(i, carry):
            pltpu.sync_copy(zbuf, acc.at[pl.ds(base + i * zrows, zrows)])
            return carry

        lax.fori_loop(0, rows_per_sub // zrows, zc_body, 0)

        pltpu.sync_copy(col_hbm.at[wid], colv)
        pltpu.sync_copy(row_hbm.at[wid], rowv)

        plsc.subcore_barrier()

        pltpu.async_copy(p_hbm.at[colv.at[0]], buf0, sem0)

        def ch_body(g, carry):
            k0 = 2 * g
            k1 = 2 * g + 1
            pltpu.make_async_copy(p_hbm.at[colv.at[k0]], buf0, sem0).wait()
            pltpu.async_copy(p_hbm.at[colv.at[k1]], buf1, sem1)
            pltpu.sync_copy(buf0, acc.at[rowv.at[k0]], add=True)
            pltpu.make_async_copy(p_hbm.at[colv.at[k1]], buf1, sem1).wait()

            @pl.when(g + 1 < _CH // 2)
            def _():
                pltpu.async_copy(p_hbm.at[colv.at[k1 + 1]], buf0, sem0)

            pltpu.sync_copy(buf1, acc.at[rowv.at[k1]], add=True)
            return carry

        lax.fori_loop(0, _CH // 2, ch_body, 0)

        plsc.subcore_barrier()

        pltpu.sync_copy(
            acc.at[pl.ds(base, rows_per_sub)],
            out_hbm.at[cid, pl.ds(base, rows_per_sub)],
        )

    return k


_sc_cache = {}


def _sc_segsum(dp):
    if dp not in _sc_cache:
        _sc_cache[dp] = _make_sc_segsum(dp)
    return _sc_cache[dp]


def _make_sc_pool():
    if "pool" in _sc_cache:
        return _sc_cache["pool"]
    k = functools.partial(
        pl.kernel,
        out_type=jax.ShapeDtypeStruct((_NW, 3, _GP, _HD), jnp.float32),
        mesh=plsc.VectorSubcoreMesh(**_sc_mesh),
        compiler_params=_sc_params,
        scratch_types=[
            pltpu.VMEM((_PC, _HD), jnp.float32),
            pltpu.VMEM((_PW,), jnp.int32),
            pltpu.VMEM((_GP, _HD), jnp.float32),
            pltpu.VMEM((_GP, _HD), jnp.float32),
            pltpu.VMEM((_GP, _HD), jnp.float32),
        ],
    )(_sc_pool_body)
    _sc_cache["pool"] = k
    return k


def _sc_pool_body(h_hbm, gid_hbm, out_hbm, chunk, gidv, accs, accn, accx):
    cid = lax.axis_index("c")
    sid = lax.axis_index("s")
    wid = sid * _NC + cid
    rbase = wid * _PW

    pltpu.sync_copy(gid_hbm.at[pl.ds(rbase, _PW)], gidv)

    def zc_body(i, carry):
        accs[i, pl.ds(_HD - 16, 16)] = jnp.zeros((16,), jnp.float32)
        return carry

    lax.fori_loop(0, _GP, zc_body, 0)

    zero = jnp.zeros((16,), jnp.float32)
    init = (jnp.int32(-1),) + tuple(zero for _ in range(3 * _HL))

    def chunk_body(c, carry):
        pltpu.sync_copy(h_hbm.at[pl.ds(rbase + c * _PC, _PC)], chunk)

        def vec_body(k, carry):
            gvec = gidv[pl.ds(c * _PC + k * 16, 16)]
            for l in range(16):
                prev = carry[0]
                ss = carry[1:1 + _HL]
                nn = carry[1 + _HL:1 + 2 * _HL]
                xx = carry[1 + 2 * _HL:]
                lane = lax.iota(jnp.int32, 16) == l
                g = jnp.max(jnp.where(lane, gvec, -1))
                is_new = g != prev

                @pl.when(jnp.logical_and(is_new, prev >= 0))
                def _():
                    for j in range(_HL):
                        accs[prev, pl.ds(j * 16, 16)] = ss[j]
                        accn[prev, pl.ds(j * 16, 16)] = nn[j]
                        accx[prev, pl.ds(j * 16, 16)] = xx[j]

                r = k * 16 + l
                vs = [chunk[r, pl.ds(j * 16, 16)] for j in range(_HL)]
                ss = tuple(jnp.where(is_new, vs[j], ss[j] + vs[j]) for j in range(_HL))
                nn = tuple(jnp.where(is_new, vs[j], jnp.minimum(nn[j], vs[j])) for j in range(_HL))
                xx = tuple(jnp.where(is_new, vs[j], jnp.maximum(xx[j], vs[j])) for j in range(_HL))
                carry = (g,) + ss + nn + xx
            return carry

        return lax.fori_loop(0, _PC // 16, vec_body, carry)

    fin = lax.fori_loop(0, _PW // _PC, chunk_body, init)
    prev = fin[0]
    for j in range(_HL):
        accs[prev, pl.ds(j * 16, 16)] = fin[1 + j]
        accn[prev, pl.ds(j * 16, 16)] = fin[1 + _HL + j]
        accx[prev, pl.ds(j * 16, 16)] = fin[1 + 2 * _HL + j]

    pltpu.sync_copy(accs, out_hbm.at[wid, 0])
    pltpu.sync_copy(accn, out_hbm.at[wid, 1])
    pltpu.sync_copy(accx, out_hbm.at[wid, 2])


def _proj_body(x_ref, nk_ref, sk_ref, pp_ref, sh_ref):
    xb = x_ref[...]
    ones = jnp.where(lax.broadcasted_iota(jnp.int32, (_R, 64), 1) == 50, 1.0, 0.0)
    pp_ref[...] = xb @ nk_ref[...] + ones
    sh_ref[...] = xb @ sk_ref[...]


def _combine_body(sp_ref, sh_ref, bn_ref, bs_ref, nkt_ref, nkb_ref, skt_ref,
                  skb_ref, pp_ref, shn_ref, *, u, uo, dpi, dpo):
    s = sp_ref[0] + sp_ref[1]
    deg = jnp.maximum(s[:, u:u + 1], 1.0)
    hm = jnp.maximum(s / deg + bn_ref[...], 0.0)
    hs = jnp.maximum(sh_ref[...] + bs_ref[...], 0.0)
    ones = jnp.where(lax.broadcasted_iota(jnp.int32, (_R, dpo), 1) == uo, 1.0, 0.0)
    pp_ref[...] = hm @ nkt_ref[...] + hs @ nkb_ref[...] + ones
    shn_ref[...] = hm @ skt_ref[...] + hs @ skb_ref[...]


def _final_body(sp_ref, sh_ref, bn_ref, bs_ref, out_ref):
    s = sp_ref[0] + sp_ref[1]
    deg = jnp.maximum(s[:, 100:101], 1.0)
    hm = jnp.maximum(s / deg + bn_ref[...], 0.0)
    hs = jnp.maximum(sh_ref[...] + bs_ref[...], 0.0)
    cnt = jnp.where(lax.broadcasted_iota(jnp.int32, (_R, 16), 1) == 0, 1.0, 0.0)
    out_ref[...] = jnp.concatenate([hm, hs, cnt], axis=1)


def _head_body(parts_ref, W1_ref, b1_ref, W2_ref, b2_ref, W3_ref, b3_ref, out_ref):
    p = parts_ref[...]
    valid = p[:, 0, :, _HD - 16:_HD - 15] > 0.0
    sums = jnp.sum(jnp.where(valid, p[:, 0], 0.0), axis=0)
    mins = jnp.min(jnp.where(valid, p[:, 1], jnp.inf), axis=0)
    maxs = jnp.max(jnp.where(valid, p[:, 2], -jnp.inf), axis=0)
    cnt = jnp.maximum(sums[:, _HD - 16:_HD - 15], 1.0)
    mean = sums / cnt
    pool = jnp.concatenate(
        [mean[:, 0:100], mean[:, 112:212], mins[:, 0:100], mins[:, 112:212],
         maxs[:, 0:100], maxs[:, 112:212], sums[:, 0:100], sums[:, 112:212]],
        axis=1,
    )
    out = pool @ W1_ref[...] + b1_ref[...]
    out = out @ W2_ref[...] + b2_ref[...]
    out = out @ W3_ref[...] + b3_ref[...]
    out_ref[...] = out[:_G]


def _row_specs(widths):
    return [pl.BlockSpec((_R, w), lambda i: (i, 0)) for w in widths]


def _full_specs(shapes):
    return [
        pl.BlockSpec(s, lambda i, _n=len(s): (0,) * _n) for s in shapes
    ]


def _pad2(a, shape, r0=0, c0=0):
    return jnp.zeros(shape, jnp.float32).at[r0:r0 + a.shape[0], c0:c0 + a.shape[1]].set(a)


def kernel(x, edge_index, node_graph_index, self_k0, neigh_k0, bias0, self_k1, neigh_k1, bias1, self_kF, neigh_kF, biasF, W1, b1, W2, b2, W3, b3):
    col3 = jnp.concatenate(
        [edge_index[1], jnp.zeros((_EP - _E,), jnp.int32)]).reshape(_NW, _CH, _C)
    pad_rows = _N + 16 + jnp.arange(_EP - _E, dtype=jnp.int32) % (_NP - _N - 16)
    row3 = jnp.concatenate([edge_index[0], pad_rows]).reshape(_NW, _CH, _C)
    xp = jnp.pad(x, ((0, _NP - _N), (0, 0)))
    gidp = jnp.concatenate([node_graph_index, jnp.full((_NP - _N,), _G, jnp.int32)])

    nk0 = _pad2(neigh_k0, (128, 64))
    sk0 = _pad2(self_k0, (128, 64))
    w1 = [_pad2(m[a:a + 50], (64, 80)) for m in (neigh_k1, self_k1) for a in (0, 50)]
    w2 = [_pad2(m[a:a + 75], (80, 112)) for m in (neigh_kF, self_kF) for a in (0, 75)]
    bn0, bs0 = _pad2(bias0[None, :50], (1, 64)), _pad2(bias0[None, 50:], (1, 64))
    bn1, bs1 = _pad2(bias1[None, :75], (1, 80)), _pad2(bias1[None, 75:], (1, 80))
    bn2, bs2 = _pad2(biasF[None, :100], (1, 112)), _pad2(biasF[None, 100:], (1, 112))

    grid = _NP // _R

    p0, sh0 = pl.pallas_call(
        _proj_body,
        grid=grid,
        in_specs=_row_specs([128]) + _full_specs([(128, 64), (128, 64)]),
        out_specs=_row_specs([64, 64]),
        out_shape=[jax.ShapeDtypeStruct((_NP, 64), jnp.float32)] * 2,
    )(xp, nk0, sk0)

    s0 = _sc_segsum(64)(p0, col3, row3)

    def combine(sp, sh, bn, bs, ws, u, uo, dpi, dpo):
        return pl.pallas_call(
            functools.partial(_combine_body, u=u, uo=uo, dpi=dpi, dpo=dpo),
            grid=grid,
            in_specs=[pl.BlockSpec((_NC, _R, dpi), lambda i: (0, i, 0))]
            + _row_specs([dpi])
            + _full_specs([(1, dpi), (1, dpi), (dpi, dpo), (dpi, dpo), (dpi, dpo), (dpi, dpo)]),
            out_specs=_row_specs([dpo, dpo]),
            out_shape=[jax.ShapeDtypeStruct((_NP, dpo), jnp.float32)] * 2,
        )(sp, sh, bn, bs, ws[0], ws[1], ws[2], ws[3])

    p1, sh1 = combine(s0, sh0, bn0, bs0, w1, 50, 75, 64, 80)
    s1 = _sc_segsum(80)(p1, col3, row3)
    p2, sh2 = combine(s1, sh1, bn1, bs1, w2, 75, 100, 80, 112)
    s2 = _sc_segsum(112)(p2, col3, row3)

    h3 = pl.pallas_call(
        _final_body,
        grid=grid,
        in_specs=[pl.BlockSpec((_NC, _R, 112), lambda i: (0, i, 0))]
        + _row_specs([112])
        + _full_specs([(1, 112), (1, 112)]),
        out_specs=pl.BlockSpec((_R, _HD), lambda i: (i, 0)),
        out_shape=jax.ShapeDtypeStruct((_NP, _HD), jnp.float32),
    )(s2, sh2, bn2, bs2)

    parts = _make_sc_pool()(h3, gidp)

    out = pl.pallas_call(
        _head_body,
        out_shape=jax.ShapeDtypeStruct((_G, 10), jnp.float32),
    )(parts, W1, b1[None], W2, b2[None], W3, b3[None])
    return out


# speedup vs baseline: 1.1137x; 1.0159x over previous
import functools

import jax
import jax.numpy as jnp
from jax import lax
from jax.experimental import pallas as pl
from jax.experimental.pallas import tpu as pltpu
from jax.experimental.pallas import tpu_sc as plsc

_N = 10000
_E = 320000
_G = 100

_NC = 2
_NS = 16
_NW = _NC * _NS
_NP = 10240
_CH = 80
_C = 125
_EP = _NW * _CH * _C

_R = 2560
_PW = _NP // _NW
_PC = 64
_GP = 104
_HD = 240
_HL = _HD // 16

_sc_mesh = dict(core_axis_name="c", subcore_axis_name="s", num_cores=_NC,
                num_subcores=_NS)
_sc_params = pltpu.CompilerParams(use_tc_tiling_on_sc=False,
                                  needs_layout_passes=False)


def _make_sc_segsum(dp):
    dl = dp // 16
    zrows = 64
    rows_per_sub = _NP // _NS

    @functools.partial(
        pl.kernel,
        out_type=jax.ShapeDtypeStruct((_NC, _NP, dp), jnp.float32),
        mesh=plsc.VectorSubcoreMesh(**_sc_mesh),
        compiler_params=_sc_params,
        scratch_types=[
            pltpu.VMEM((_CH, _C), jnp.int32),
            pltpu.VMEM((_CH, _C), jnp.int32),
            pltpu.VMEM((_C, dp), jnp.float32),
            pltpu.VMEM((_C, dp), jnp.float32),
            pltpu.VMEM((zrows, dp), jnp.float32),
            pltpu.VMEM_SHARED((_NP, dp), jnp.float32),
            pltpu.SemaphoreType.DMA,
            pltpu.SemaphoreType.DMA,
        ],
    )
    def k(p_hbm, col_hbm, row_hbm, out_hbm, colv, rowv, buf0, buf1, zbuf, acc, sem0, sem1):
        cid = lax.axis_index("c")
        sid = lax.axis_index("s")
        wid = sid * _NC + cid

        def zb_body(i, carry):
            r = i // dl
            j = i - r * dl
            zbuf[r, pl.ds(j * 16, 16)] = jnp.zeros((16,), jnp.float32)
            return carry

        lax.fori_loop(0, zrows * dl, zb_body, 0)

        base = sid * rows_per_sub

        def zc_body(i, carry):
            pltpu.sync_copy(zbuf, acc.at[pl.ds(base + i * zrows, zrows)])
            return carry

        lax.fori_loop(0, rows_per_sub // zrows, zc_body, 0)

        pltpu.sync_copy(col_hbm.at[wid], colv)
        pltpu.sync_copy(row_hbm.at[wid], rowv)

        plsc.subcore_barrier()

        pltpu.async_copy(p_hbm.at[colv.at[0]], buf0, sem0)

        def ch_body(g, carry):
            k0 = 2 * g
            k1 = 2 * g + 1
            pltpu.make_async_copy(p_hbm.at[colv.at[k0]], buf0, sem0).wait()
            pltpu.async_copy(p_hbm.at[colv.at[k1]], buf1, sem1)
            pltpu.sync_copy(buf0, acc.at[rowv.at[k0]], add=True)
            pltpu.make_async_copy(p_hbm.at[colv.at[k1]], buf1, sem1).wait()

            @pl.when(g + 1 < _CH // 2)
            def _():
                pltpu.async_copy(p_hbm.at[colv.at[k1 + 1]], buf0, sem0)

            pltpu.sync_copy(buf1, acc.at[rowv.at[k1]], add=True)
            return carry

        lax.fori_loop(0, _CH // 2, ch_body, 0)

        plsc.subcore_barrier()

        pltpu.sync_copy(
            acc.at[pl.ds(base, rows_per_sub)],
            out_hbm.at[cid, pl.ds(base, rows_per_sub)],
        )

    return k


_sc_cache = {}


def _sc_segsum(dp):
    if dp not in _sc_cache:
        _sc_cache[dp] = _make_sc_segsum(dp)
    return _sc_cache[dp]


def _make_sc_pool():
    if "pool" in _sc_cache:
        return _sc_cache["pool"]
    k = functools.partial(
        pl.kernel,
        out_type=jax.ShapeDtypeStruct((_NW, 3, _GP, _HD), jnp.float32),
        mesh=plsc.VectorSubcoreMesh(**_sc_mesh),
        compiler_params=_sc_params,
        scratch_types=[
            pltpu.VMEM((_PC, _HD), jnp.float32),
            pltpu.VMEM((_PW,), jnp.int32),
            pltpu.VMEM((_GP, _HD), jnp.float32),
            pltpu.VMEM((_GP, _HD), jnp.float32),
            pltpu.VMEM((_GP, _HD), jnp.float32),
        ],
    )(_sc_pool_body)
    _sc_cache["pool"] = k
    return k


def _sc_pool_body(h_hbm, gid_hbm, out_hbm, chunk, gidv, accs, accn, accx):
    cid = lax.axis_index("c")
    sid = lax.axis_index("s")
    wid = sid * _NC + cid
    rbase = wid * _PW

    pltpu.sync_copy(gid_hbm.at[pl.ds(rbase, _PW)], gidv)

    def zc_body(i, carry):
        accs[i, pl.ds(_HD - 16, 16)] = jnp.zeros((16,), jnp.float32)
        return carry

    lax.fori_loop(0, _GP, zc_body, 0)

    zero = jnp.zeros((16,), jnp.float32)
    init = (jnp.int32(-1),) + tuple(zero for _ in range(3 * _HL))

    def chunk_body(c, carry):
        pltpu.sync_copy(h_hbm.at[pl.ds(rbase + c * _PC, _PC)], chunk)

        def vec_body(k, carry):
            gvec = gidv[pl.ds(c * _PC + k * 16, 16)]
            for l in range(16):
                prev = carry[0]
                ss = carry[1:1 + _HL]
                nn = carry[1 + _HL:1 + 2 * _HL]
                xx = carry[1 + 2 * _HL:]
                lane = lax.iota(jnp.int32, 16) == l
                g = jnp.max(jnp.where(lane, gvec, -1))
                is_new = g != prev

                @pl.when(jnp.logical_and(is_new, prev >= 0))
                def _():
                    for j in range(_HL):
                        accs[prev, pl.ds(j * 16, 16)] = ss[j]
                        accn[prev, pl.ds(j * 16, 16)] = nn[j]
                        accx[prev, pl.ds(j * 16, 16)] = xx[j]

                r = k * 16 + l
                vs = [chunk[r, pl.ds(j * 16, 16)] for j in range(_HL)]
                ss = tuple(jnp.where(is_new, vs[j], ss[j] + vs[j]) for j in range(_HL))
                nn = tuple(jnp.where(is_new, vs[j], jnp.minimum(nn[j], vs[j])) for j in range(_HL))
                xx = tuple(jnp.where(is_new, vs[j], jnp.maximum(xx[j], vs[j])) for j in range(_HL))
                carry = (g,) + ss + nn + xx
            return carry

        return lax.fori_loop(0, _PC // 16, vec_body, carry)

    fin = lax.fori_loop(0, _PW // _PC, chunk_body, init)
    prev = fin[0]
    for j in range(_HL):
        accs[prev, pl.ds(j * 16, 16)] = fin[1 + j]
        accn[prev, pl.ds(j * 16, 16)] = fin[1 + _HL + j]
        accx[prev, pl.ds(j * 16, 16)] = fin[1 + 2 * _HL + j]

    pltpu.sync_copy(accs, out_hbm.at[wid, 0])
    pltpu.sync_copy(accn, out_hbm.at[wid, 1])
    pltpu.sync_copy(accx, out_hbm.at[wid, 2])


def _proj_body(x_ref, nk_ref, sk_ref, pp_ref, sh_ref):
    xb = x_ref[...]
    ones = jnp.where(lax.broadcasted_iota(jnp.int32, (_R, 64), 1) == 50, 1.0, 0.0)
    pp_ref[...] = xb @ nk_ref[...] + ones
    sh_ref[...] = xb @ sk_ref[...]


def _combine_body(sp_ref, sh_ref, bn_ref, bs_ref, nkt_ref, nkb_ref, skt_ref,
                  skb_ref, pp_ref, shn_ref, *, u, uo, dpi, dpo):
    s = sp_ref[0] + sp_ref[1]
    deg = jnp.maximum(s[:, u:u + 1], 1.0)
    hm = jnp.maximum(s / deg + bn_ref[...], 0.0)
    hs = jnp.maximum(sh_ref[...] + bs_ref[...], 0.0)
    ones = jnp.where(lax.broadcasted_iota(jnp.int32, (_R, dpo), 1) == uo, 1.0, 0.0)
    pp_ref[...] = hm @ nkt_ref[...] + hs @ nkb_ref[...] + ones
    shn_ref[...] = hm @ skt_ref[...] + hs @ skb_ref[...]


def _final_body(sp_ref, sh_ref, bn_ref, bs_ref, out_ref):
    s = sp_ref[0] + sp_ref[1]
    deg = jnp.maximum(s[:, 100:101], 1.0)
    hm = jnp.maximum(s / deg + bn_ref[...], 0.0)
    hs = jnp.maximum(sh_ref[...] + bs_ref[...], 0.0)
    cnt = jnp.where(lax.broadcasted_iota(jnp.int32, (_R, 16), 1) == 0, 1.0, 0.0)
    out_ref[...] = jnp.concatenate([hm, hs, cnt], axis=1)


def _head_body(parts_ref, W1_ref, b1_ref, W2_ref, b2_ref, W3_ref, b3_ref, out_ref):
    p = parts_ref[...]
    valid = p[:, 0, :, _HD - 16:_HD - 15] > 0.0
    sums = jnp.sum(jnp.where(valid, p[:, 0], 0.0), axis=0)
    mins = jnp.min(jnp.where(valid, p[:, 1], jnp.inf), axis=0)
    maxs = jnp.max(jnp.where(valid, p[:, 2], -jnp.inf), axis=0)
    cnt = jnp.maximum(sums[:, _HD - 16:_HD - 15], 1.0)
    mean = sums / cnt
    pool = jnp.concatenate(
        [mean[:, 0:100], mean[:, 112:212], mins[:, 0:100], mins[:, 112:212],
         maxs[:, 0:100], maxs[:, 112:212], sums[:, 0:100], sums[:, 112:212]],
        axis=1,
    )
    out = pool @ W1_ref[...] + b1_ref[...]
    out = out @ W2_ref[...] + b2_ref[...]
    out = out @ W3_ref[...] + b3_ref[...]
    out_ref[...] = out[:_G]


def _row_specs(widths):
    return [pl.BlockSpec((_R, w), lambda i: (i, 0)) for w in widths]


def _full_specs(shapes):
    return [
        pl.BlockSpec(s, lambda i, _n=len(s): (0,) * _n) for s in shapes
    ]


def _pad2(a, shape, r0=0, c0=0):
    return jnp.zeros(shape, jnp.float32).at[r0:r0 + a.shape[0], c0:c0 + a.shape[1]].set(a)


def kernel(x, edge_index, node_graph_index, self_k0, neigh_k0, bias0, self_k1, neigh_k1, bias1, self_kF, neigh_kF, biasF, W1, b1, W2, b2, W3, b3):
    col3 = jnp.concatenate(
        [edge_index[1], jnp.zeros((_EP - _E,), jnp.int32)]).reshape(_NW, _CH, _C)
    pad_rows = _N + 16 + jnp.arange(_EP - _E, dtype=jnp.int32) % (_NP - _N - 16)
    row3 = jnp.concatenate([edge_index[0], pad_rows]).reshape(_NW, _CH, _C)
    xp = jnp.pad(x, ((0, _NP - _N), (0, 0)))
    gidp = jnp.concatenate([node_graph_index, jnp.full((_NP - _N,), _G, jnp.int32)])

    nk0 = _pad2(neigh_k0, (128, 64))
    sk0 = _pad2(self_k0, (128, 64))
    w1 = [_pad2(m[a:a + 50], (64, 80)) for m in (neigh_k1, self_k1) for a in (0, 50)]
    w2 = [_pad2(m[a:a + 75], (80, 112)) for m in (neigh_kF, self_kF) for a in (0, 75)]
    bn0, bs0 = _pad2(bias0[None, :50], (1, 64)), _pad2(bias0[None, 50:], (1, 64))
    bn1, bs1 = _pad2(bias1[None, :75], (1, 80)), _pad2(bias1[None, 75:], (1, 80))
    bn2, bs2 = _pad2(biasF[None, :100], (1, 112)), _pad2(biasF[None, 100:], (1, 112))

    grid = _NP // _R

    p0, sh0 = pl.pallas_call(
        _proj_body,
        grid=grid,
        in_specs=_row_specs([128]) + _full_specs([(128, 64), (128, 64)]),
        out_specs=_row_specs([64, 64]),
        out_shape=[jax.ShapeDtypeStruct((_NP, 64), jnp.float32)] * 2,
    )(xp, nk0, sk0)

    s0 = _sc_segsum(64)(p0, col3, row3)

    def combine(sp, sh, bn, bs, ws, u, uo, dpi, dpo):
        return pl.pallas_call(
            functools.partial(_combine_body, u=u, uo=uo, dpi=dpi, dpo=dpo),
            grid=grid,
            in_specs=[pl.BlockSpec((_NC, _R, dpi), lambda i: (0, i, 0))]
            + _row_specs([dpi])
            + _full_specs([(1, dpi), (1, dpi), (dpi, dpo), (dpi, dpo), (dpi, dpo), (dpi, dpo)]),
            out_specs=_row_specs([dpo, dpo]),
            out_shape=[jax.ShapeDtypeStruct((_NP, dpo), jnp.float32)] * 2,
        )(sp, sh, bn, bs, ws[0], ws[1], ws[2], ws[3])

    p1, sh1 = combine(s0, sh0, bn0, bs0, w1, 50, 75, 64, 80)
    s1 = _sc_segsum(80)(p1, col3, row3)
    p2, sh2 = combine(s1, sh1, bn1, bs1, w2, 75, 100, 80, 112)
    s2 = _sc_segsum(112)(p2, col3, row3)

    h3 = pl.pallas_call(
        _final_body,
        grid=grid,
        in_specs=[pl.BlockSpec((_NC, _R, 112), lambda i: (0, i, 0))]
        + _row_specs([112])
        + _full_specs([(1, 112), (1, 112)]),
        out_specs=pl.BlockSpec((_R, _HD), lambda i: (i, 0)),
        out_shape=jax.ShapeDtypeStruct((_NP, _HD), jnp.float32),
    )(s2, sh2, bn2, bs2)

    parts = _make_sc_pool()(h3, gidp)

    out = pl.pallas_call(
        _head_body,
        out_shape=jax.ShapeDtypeStruct((_G, 10), jnp.float32),
    )(parts, W1, b1[None], W2, b2[None], W3, b3[None])
    return out


# speedup vs baseline: 1.1256x; 1.0107x over previous
import functools

import jax
import jax.numpy as jnp
from jax import lax
from jax.experimental import pallas as pl
from jax.experimental.pallas import tpu as pltpu
from jax.experimental.pallas import tpu_sc as plsc

_N = 10000
_E = 320000
_G = 100

_NC = 2
_NS = 16
_NW = _NC * _NS
_NP = 10240
_CH = 80
_C = 125
_EP = _NW * _CH * _C

_R = 5120
_PW = _NP // _NW
_PC = 64
_GP = 104
_HD = 240
_HL = _HD // 16

_sc_mesh = dict(core_axis_name="c", subcore_axis_name="s", num_cores=_NC,
                num_subcores=_NS)
_sc_params = pltpu.CompilerParams(use_tc_tiling_on_sc=False,
                                  needs_layout_passes=False)


def _make_sc_segsum(dp):
    dl = dp // 16
    zrows = 64
    rows_per_sub = _NP // _NS

    @functools.partial(
        pl.kernel,
        out_type=jax.ShapeDtypeStruct((_NC, _NP, dp), jnp.float32),
        mesh=plsc.VectorSubcoreMesh(**_sc_mesh),
        compiler_params=_sc_params,
        scratch_types=[
            pltpu.VMEM((_CH, _C), jnp.int32),
            pltpu.VMEM((_CH, _C), jnp.int32),
            pltpu.VMEM((_C, dp), jnp.float32),
            pltpu.VMEM((_C, dp), jnp.float32),
            pltpu.VMEM((zrows, dp), jnp.float32),
            pltpu.VMEM_SHARED((_NP, dp), jnp.float32),
            pltpu.SemaphoreType.DMA,
            pltpu.SemaphoreType.DMA,
        ],
    )
    def k(p_hbm, col_hbm, row_hbm, out_hbm, colv, rowv, buf0, buf1, zbuf, acc, sem0, sem1):
        cid = lax.axis_index("c")
        sid = lax.axis_index("s")
        wid = sid * _NC + cid

        def zb_body(i, carry):
            r = i // dl
            j = i - r * dl
            zbuf[r, pl.ds(j * 16, 16)] = jnp.zeros((16,), jnp.float32)
            return carry

        lax.fori_loop(0, zrows * dl, zb_body, 0)

        base = sid * rows_per_sub

        def zc_body(i, carry):
            pltpu.sync_copy(zbuf, acc.at[pl.ds(base + i * zrows, zrows)])
            return carry

        lax.fori_loop(0, rows_per_sub // zrows, zc_body, 0)

        pltpu.sync_copy(col_hbm.at[wid], colv)
        pltpu.sync_copy(row_hbm.at[wid], rowv)

        plsc.subcore_barrier()

        pltpu.async_copy(p_hbm.at[colv.at[0]], buf0, sem0)

        def ch_body(g, carry):
            k0 = 2 * g
            k1 = 2 * g + 1
            pltpu.make_async_copy(p_hbm.at[colv.at[k0]], buf0, sem0).wait()
            pltpu.async_copy(p_hbm.at[colv.at[k1]], buf1, sem1)
            pltpu.sync_copy(buf0, acc.at[rowv.at[k0]], add=True)
            pltpu.make_async_copy(p_hbm.at[colv.at[k1]], buf1, sem1).wait()

            @pl.when(g + 1 < _CH // 2)
            def _():
                pltpu.async_copy(p_hbm.at[colv.at[k1 + 1]], buf0, sem0)

            pltpu.sync_copy(buf1, acc.at[rowv.at[k1]], add=True)
            return carry

        lax.fori_loop(0, _CH // 2, ch_body, 0)

        plsc.subcore_barrier()

        pltpu.sync_copy(
            acc.at[pl.ds(base, rows_per_sub)],
            out_hbm.at[cid, pl.ds(base, rows_per_sub)],
        )

    return k


_sc_cache = {}


def _sc_segsum(dp):
    if dp not in _sc_cache:
        _sc_cache[dp] = _make_sc_segsum(dp)
    return _sc_cache[dp]


def _make_sc_pool():
    if "pool" in _sc_cache:
        return _sc_cache["pool"]
    k = functools.partial(
        pl.kernel,
        out_type=jax.ShapeDtypeStruct((_NW, 3, _GP, _HD), jnp.float32),
        mesh=plsc.VectorSubcoreMesh(**_sc_mesh),
        compiler_params=_sc_params,
        scratch_types=[
            pltpu.VMEM((_PC, _HD), jnp.float32),
            pltpu.VMEM((_PW,), jnp.int32),
            pltpu.VMEM((_GP, _HD), jnp.float32),
            pltpu.VMEM((_GP, _HD), jnp.float32),
            pltpu.VMEM((_GP, _HD), jnp.float32),
        ],
    )(_sc_pool_body)
    _sc_cache["pool"] = k
    return k


def _sc_pool_body(h_hbm, gid_hbm, out_hbm, chunk, gidv, accs, accn, accx):
    cid = lax.axis_index("c")
    sid = lax.axis_index("s")
    wid = sid * _NC + cid
    rbase = wid * _PW

    pltpu.sync_copy(gid_hbm.at[pl.ds(rbase, _PW)], gidv)

    def zc_body(i, carry):
        accs[i, pl.ds(_HD - 16, 16)] = jnp.zeros((16,), jnp.float32)
        return carry

    lax.fori_loop(0, _GP, zc_body, 0)

    zero = jnp.zeros((16,), jnp.float32)
    init = (jnp.int32(-1),) + tuple(zero for _ in range(3 * _HL))

    def chunk_body(c, carry):
        pltpu.sync_copy(h_hbm.at[pl.ds(rbase + c * _PC, _PC)], chunk)

        def vec_body(k, carry):
            gvec = gidv[pl.ds(c * _PC + k * 16, 16)]
            for l in range(16):
                prev = carry[0]
                ss = carry[1:1 + _HL]
                nn = carry[1 + _HL:1 + 2 * _HL]
                xx = carry[1 + 2 * _HL:]
                lane = lax.iota(jnp.int32, 16) == l
                g = jnp.max(jnp.where(lane, gvec, -1))
                is_new = g != prev

                @pl.when(jnp.logical_and(is_new, prev >= 0))
                def _():
                    for j in range(_HL):
                        accs[prev, pl.ds(j * 16, 16)] = ss[j]
                        accn[prev, pl.ds(j * 16, 16)] = nn[j]
                        accx[prev, pl.ds(j * 16, 16)] = xx[j]

                r = k * 16 + l
                vs = [chunk[r, pl.ds(j * 16, 16)] for j in range(_HL)]
                ss = tuple(jnp.where(is_new, vs[j], ss[j] + vs[j]) for j in range(_HL))
                nn = tuple(jnp.where(is_new, vs[j], jnp.minimum(nn[j], vs[j])) for j in range(_HL))
                xx = tuple(jnp.where(is_new, vs[j], jnp.maximum(xx[j], vs[j])) for j in range(_HL))
                carry = (g,) + ss + nn + xx
            return carry

        return lax.fori_loop(0, _PC // 16, vec_body, carry)

    fin = lax.fori_loop(0, _PW // _PC, chunk_body, init)
    prev = fin[0]
    for j in range(_HL):
        accs[prev, pl.ds(j * 16, 16)] = fin[1 + j]
        accn[prev, pl.ds(j * 16, 16)] = fin[1 + _HL + j]
        accx[prev, pl.ds(j * 16, 16)] = fin[1 + 2 * _HL + j]

    pltpu.sync_copy(accs, out_hbm.at[wid, 0])
    pltpu.sync_copy(accn, out_hbm.at[wid, 1])
    pltpu.sync_copy(accx, out_hbm.at[wid, 2])


def _proj_body(x_ref, nk_ref, sk_ref, pp_ref, sh_ref):
    xb = x_ref[...]
    ones = jnp.where(lax.broadcasted_iota(jnp.int32, (_R, 64), 1) == 50, 1.0, 0.0)
    pp_ref[...] = xb @ nk_ref[...] + ones
    sh_ref[...] = xb @ sk_ref[...]


def _combine_body(sp_ref, sh_ref, bn_ref, bs_ref, nkt_ref, nkb_ref, skt_ref,
                  skb_ref, pp_ref, shn_ref, *, u, uo, dpi, dpo):
    s = sp_ref[0] + sp_ref[1]
    deg = jnp.maximum(s[:, u:u + 1], 1.0)
    hm = jnp.maximum(s / deg + bn_ref[...], 0.0)
    hs = jnp.maximum(sh_ref[...] + bs_ref[...], 0.0)
    ones = jnp.where(lax.broadcasted_iota(jnp.int32, (_R, dpo), 1) == uo, 1.0, 0.0)
    pp_ref[...] = hm @ nkt_ref[...] + hs @ nkb_ref[...] + ones
    shn_ref[...] = hm @ skt_ref[...] + hs @ skb_ref[...]


def _final_body(sp_ref, sh_ref, bn_ref, bs_ref, out_ref):
    s = sp_ref[0] + sp_ref[1]
    deg = jnp.maximum(s[:, 100:101], 1.0)
    hm = jnp.maximum(s / deg + bn_ref[...], 0.0)
    hs = jnp.maximum(sh_ref[...] + bs_ref[...], 0.0)
    cnt = jnp.where(lax.broadcasted_iota(jnp.int32, (_R, 16), 1) == 0, 1.0, 0.0)
    out_ref[...] = jnp.concatenate([hm, hs, cnt], axis=1)


def _head_body(parts_ref, W1_ref, b1_ref, W2_ref, b2_ref, W3_ref, b3_ref, out_ref):
    p = parts_ref[...]
    valid = p[:, 0, :, _HD - 16:_HD - 15] > 0.0
    sums = jnp.sum(jnp.where(valid, p[:, 0], 0.0), axis=0)
    mins = jnp.min(jnp.where(valid, p[:, 1], jnp.inf), axis=0)
    maxs = jnp.max(jnp.where(valid, p[:, 2], -jnp.inf), axis=0)
    cnt = jnp.maximum(sums[:, _HD - 16:_HD - 15], 1.0)
    mean = sums / cnt
    pool = jnp.concatenate(
        [mean[:, 0:100], mean[:, 112:212], mins[:, 0:100], mins[:, 112:212],
         maxs[:, 0:100], maxs[:, 112:212], sums[:, 0:100], sums[:, 112:212]],
        axis=1,
    )
    out = pool @ W1_ref[...] + b1_ref[...]
    out = out @ W2_ref[...] + b2_ref[...]
    out = out @ W3_ref[...] + b3_ref[...]
    out_ref[...] = out[:_G]


def _row_specs(widths):
    return [pl.BlockSpec((_R, w), lambda i: (i, 0)) for w in widths]


def _full_specs(shapes):
    return [
        pl.BlockSpec(s, lambda i, _n=len(s): (0,) * _n) for s in shapes
    ]


def _pad2(a, shape, r0=0, c0=0):
    return jnp.zeros(shape, jnp.float32).at[r0:r0 + a.shape[0], c0:c0 + a.shape[1]].set(a)


def kernel(x, edge_index, node_graph_index, self_k0, neigh_k0, bias0, self_k1, neigh_k1, bias1, self_kF, neigh_kF, biasF, W1, b1, W2, b2, W3, b3):
    col3 = jnp.concatenate(
        [edge_index[1], jnp.zeros((_EP - _E,), jnp.int32)]).reshape(_NW, _CH, _C)
    pad_rows = _N + 16 + jnp.arange(_EP - _E, dtype=jnp.int32) % (_NP - _N - 16)
    row3 = jnp.concatenate([edge_index[0], pad_rows]).reshape(_NW, _CH, _C)
    xp = jnp.pad(x, ((0, _NP - _N), (0, 0)))
    gidp = jnp.concatenate([node_graph_index, jnp.full((_NP - _N,), _G, jnp.int32)])

    nk0 = _pad2(neigh_k0, (128, 64))
    sk0 = _pad2(self_k0, (128, 64))
    w1 = [_pad2(m[a:a + 50], (64, 80)) for m in (neigh_k1, self_k1) for a in (0, 50)]
    w2 = [_pad2(m[a:a + 75], (80, 112)) for m in (neigh_kF, self_kF) for a in (0, 75)]
    bn0, bs0 = _pad2(bias0[None, :50], (1, 64)), _pad2(bias0[None, 50:], (1, 64))
    bn1, bs1 = _pad2(bias1[None, :75], (1, 80)), _pad2(bias1[None, 75:], (1, 80))
    bn2, bs2 = _pad2(biasF[None, :100], (1, 112)), _pad2(biasF[None, 100:], (1, 112))

    grid = _NP // _R

    p0, sh0 = pl.pallas_call(
        _proj_body,
        grid=grid,
        in_specs=_row_specs([128]) + _full_specs([(128, 64), (128, 64)]),
        out_specs=_row_specs([64, 64]),
        out_shape=[jax.ShapeDtypeStruct((_NP, 64), jnp.float32)] * 2,
    )(xp, nk0, sk0)

    s0 = _sc_segsum(64)(p0, col3, row3)

    def combine(sp, sh, bn, bs, ws, u, uo, dpi, dpo):
        return pl.pallas_call(
            functools.partial(_combine_body, u=u, uo=uo, dpi=dpi, dpo=dpo),
            grid=grid,
            in_specs=[pl.BlockSpec((_NC, _R, dpi), lambda i: (0, i, 0))]
            + _row_specs([dpi])
            + _full_specs([(1, dpi), (1, dpi), (dpi, dpo), (dpi, dpo), (dpi, dpo), (dpi, dpo)]),
            out_specs=_row_specs([dpo, dpo]),
            out_shape=[jax.ShapeDtypeStruct((_NP, dpo), jnp.float32)] * 2,
        )(sp, sh, bn, bs, ws[0], ws[1], ws[2], ws[3])

    p1, sh1 = combine(s0, sh0, bn0, bs0, w1, 50, 75, 64, 80)
    s1 = _sc_segsum(80)(p1, col3, row3)
    p2, sh2 = combine(s1, sh1, bn1, bs1, w2, 75, 100, 80, 112)
    s2 = _sc_segsum(112)(p2, col3, row3)

    h3 = pl.pallas_call(
        _final_body,
        grid=grid,
        in_specs=[pl.BlockSpec((_NC, _R, 112), lambda i: (0, i, 0))]
        + _row_specs([112])
        + _full_specs([(1, 112), (1, 112)]),
        out_specs=pl.BlockSpec((_R, _HD), lambda i: (i, 0)),
        out_shape=jax.ShapeDtypeStruct((_NP, _HD), jnp.float32),
    )(s2, sh2, bn2, bs2)

    parts = _make_sc_pool()(h3, gidp)

    out = pl.pallas_call(
        _head_body,
        out_shape=jax.ShapeDtypeStruct((_G, 10), jnp.float32),
    )(parts, W1, b1[None], W2, b2[None], W3, b3[None])
    return out


# speedup vs baseline: 1.1344x; 1.0078x over previous
import functools

import jax
import jax.numpy as jnp
from jax import lax
from jax.experimental import pallas as pl
from jax.experimental.pallas import tpu as pltpu
from jax.experimental.pallas import tpu_sc as plsc

_N = 10000
_E = 320000
_G = 100

_NC = 2
_NS = 16
_NW = _NC * _NS
_NP = 10240
_CH = 80
_C = 125
_EP = _NW * _CH * _C

_R = 5120
_PW = _NP // _NW
_PC = 64
_GP = 104
_HD = 240
_HL = _HD // 16

_sc_mesh = dict(core_axis_name="c", subcore_axis_name="s", num_cores=_NC,
                num_subcores=_NS)
_sc_params = pltpu.CompilerParams(use_tc_tiling_on_sc=False,
                                  needs_layout_passes=False)


def _make_sc_segsum(dp):
    dl = dp // 16
    zrows = 64
    rows_per_sub = _NP // _NS

    @functools.partial(
        pl.kernel,
        out_type=jax.ShapeDtypeStruct((_NC, _NP, dp), jnp.float32),
        mesh=plsc.VectorSubcoreMesh(**_sc_mesh),
        compiler_params=_sc_params,
        scratch_types=[
            pltpu.VMEM((_CH, _C), jnp.int32),
            pltpu.VMEM((_CH, _C), jnp.int32),
            pltpu.VMEM((_C, dp), jnp.float32),
            pltpu.VMEM((_C, dp), jnp.float32),
            pltpu.VMEM((zrows, dp), jnp.float32),
            pltpu.VMEM_SHARED((_NP, dp), jnp.float32),
            pltpu.SemaphoreType.DMA,
            pltpu.SemaphoreType.DMA,
        ],
    )
    def k(p_hbm, col_hbm, row_hbm, out_hbm, colv, rowv, buf0, buf1, zbuf, acc, sem0, sem1):
        cid = lax.axis_index("c")
        sid = lax.axis_index("s")
        wid = sid * _NC + cid

        def zb_body(i, carry):
            r = i // dl
            j = i - r * dl
            zbuf[r, pl.ds(j * 16, 16)] = jnp.zeros((16,), jnp.float32)
            return carry

        lax.fori_loop(0, zrows * dl, zb_body, 0)

        base = sid * rows_per_sub

        def zc_body(i, carry):
            pltpu.sync_copy(zbuf, acc.at[pl.ds(base + i * zrows, zrows)])
            return carry

        lax.fori_loop(0, rows_per_sub // zrows, zc_body, 0)

        pltpu.sync_copy(col_hbm.at[wid], colv)
        pltpu.sync_copy(row_hbm.at[wid], rowv)

        plsc.subcore_barrier()

        pltpu.async_copy(p_hbm.at[colv.at[0]], buf0, sem0)

        def ch_body(g, carry):
            k0 = 2 * g
            k1 = 2 * g + 1
            pltpu.make_async_copy(p_hbm.at[colv.at[k0]], buf0, sem0).wait()
            pltpu.async_copy(p_hbm.at[colv.at[k1]], buf1, sem1)
            pltpu.sync_copy(buf0, acc.at[rowv.at[k0]], add=True)
            pltpu.make_async_copy(p_hbm.at[colv.at[k1]], buf1, sem1).wait()

            @pl.when(g + 1 < _CH // 2)
            def _():
                pltpu.async_copy(p_hbm.at[colv.at[k1 + 1]], buf0, sem0)

            pltpu.sync_copy(buf1, acc.at[rowv.at[k1]], add=True)
            return carry

        lax.fori_loop(0, _CH // 2, ch_body, 0)

        plsc.subcore_barrier()

        pltpu.sync_copy(
            acc.at[pl.ds(base, rows_per_sub)],
            out_hbm.at[cid, pl.ds(base, rows_per_sub)],
        )

    return k


_sc_cache = {}


def _sc_segsum(dp):
    if dp not in _sc_cache:
        _sc_cache[dp] = _make_sc_segsum(dp)
    return _sc_cache[dp]


def _make_sc_pool():
    if "pool" in _sc_cache:
        return _sc_cache["pool"]
    k = functools.partial(
        pl.kernel,
        out_type=jax.ShapeDtypeStruct((_NW, 3, _GP, _HD), jnp.float32),
        mesh=plsc.VectorSubcoreMesh(**_sc_mesh),
        compiler_params=_sc_params,
        scratch_types=[
            pltpu.VMEM((_PC, _HD), jnp.float32),
            pltpu.VMEM((_PW,), jnp.int32),
            pltpu.VMEM((_GP, _HD), jnp.float32),
            pltpu.VMEM((_GP, _HD), jnp.float32),
            pltpu.VMEM((_GP, _HD), jnp.float32),
        ],
    )(_sc_pool_body)
    _sc_cache["pool"] = k
    return k


def _sc_pool_body(h_hbm, gid_hbm, out_hbm, chunk, gidv, accs, accn, accx):
    cid = lax.axis_index("c")
    sid = lax.axis_index("s")
    wid = sid * _NC + cid
    rbase = wid * _PW

    pltpu.sync_copy(gid_hbm.at[pl.ds(rbase, _PW)], gidv)

    def zc_body(i, carry):
        accs[i, pl.ds(_HD - 16, 16)] = jnp.zeros((16,), jnp.float32)
        return carry

    lax.fori_loop(0, _GP, zc_body, 0)

    zero = jnp.zeros((16,), jnp.float32)
    init = (jnp.int32(-1),) + tuple(zero for _ in range(3 * _HL))

    def chunk_body(c, carry):
        pltpu.sync_copy(h_hbm.at[pl.ds(rbase + c * _PC, _PC)], chunk)

        def vec_body(k, carry):
            gvec = gidv[pl.ds(c * _PC + k * 16, 16)]
            for l in range(16):
                prev = carry[0]
                ss = carry[1:1 + _HL]
                nn = carry[1 + _HL:1 + 2 * _HL]
                xx = carry[1 + 2 * _HL:]
                lane = lax.iota(jnp.int32, 16) == l
                g = jnp.max(jnp.where(lane, gvec, -1))
                is_new = g != prev

                @pl.when(jnp.logical_and(is_new, prev >= 0))
                def _():
                    for j in range(_HL):
                        accs[prev, pl.ds(j * 16, 16)] = ss[j]
                        accn[prev, pl.ds(j * 16, 16)] = nn[j]
                        accx[prev, pl.ds(j * 16, 16)] = xx[j]

                r = k * 16 + l
                vs = [chunk[r, pl.ds(j * 16, 16)] for j in range(_HL)]
                ss = tuple(jnp.where(is_new, vs[j], ss[j] + vs[j]) for j in range(_HL))
                nn = tuple(jnp.where(is_new, vs[j], jnp.minimum(nn[j], vs[j])) for j in range(_HL))
                xx = tuple(jnp.where(is_new, vs[j], jnp.maximum(xx[j], vs[j])) for j in range(_HL))
                carry = (g,) + ss + nn + xx
            return carry

        return lax.fori_loop(0, _PC // 16, vec_body, carry)

    fin = lax.fori_loop(0, _PW // _PC, chunk_body, init)
    prev = fin[0]
    for j in range(_HL):
        accs[prev, pl.ds(j * 16, 16)] = fin[1 + j]
        accn[prev, pl.ds(j * 16, 16)] = fin[1 + _HL + j]
        accx[prev, pl.ds(j * 16, 16)] = fin[1 + 2 * _HL + j]

    pltpu.sync_copy(accs, out_hbm.at[wid, 0])
    pltpu.sync_copy(accn, out_hbm.at[wid, 1])
    pltpu.sync_copy(accx, out_hbm.at[wid, 2])


def _proj_body(x_ref, nk_ref, sk_ref, pp_ref, sh_ref):
    xb = x_ref[...]
    ones = jnp.where(lax.broadcasted_iota(jnp.int32, (_R, 64), 1) == 50, 1.0, 0.0)
    pp_ref[...] = xb @ nk_ref[...] + ones
    sh_ref[...] = xb @ sk_ref[...]


def _combine_body(sp_ref, sh_ref, bn_ref, bs_ref, nkt_ref, nkb_ref, skt_ref,
                  skb_ref, pp_ref, shn_ref, *, u, uo, dpi, dpo):
    s = sp_ref[0] + sp_ref[1]
    deg = jnp.maximum(s[:, u:u + 1], 1.0)
    hm = jnp.maximum(s / deg + bn_ref[...], 0.0)
    hs = jnp.maximum(sh_ref[...] + bs_ref[...], 0.0)
    ones = jnp.where(lax.broadcasted_iota(jnp.int32, (_R, dpo), 1) == uo, 1.0, 0.0)
    pp_ref[...] = hm @ nkt_ref[...] + hs @ nkb_ref[...] + ones
    shn_ref[...] = hm @ skt_ref[...] + hs @ skb_ref[...]


def _final_body(sp_ref, sh_ref, bn_ref, bs_ref, out_ref):
    s = sp_ref[0] + sp_ref[1]
    deg = jnp.maximum(s[:, 100:101], 1.0)
    hm = jnp.maximum(s / deg + bn_ref[...], 0.0)
    hs = jnp.maximum(sh_ref[...] + bs_ref[...], 0.0)
    cnt = jnp.where(lax.broadcasted_iota(jnp.int32, (_R, 16), 1) == 0, 1.0, 0.0)
    out_ref[...] = jnp.concatenate([hm, hs, cnt], axis=1)


def _head_body(parts_ref, W1_ref, b1_ref, W2_ref, b2_ref, W3_ref, b3_ref, out_ref):
    p = parts_ref[...]
    valid = p[:, 0, :, _HD - 16:_HD - 15] > 0.0
    sums = jnp.sum(jnp.where(valid, p[:, 0], 0.0), axis=0)
    mins = jnp.min(jnp.where(valid, p[:, 1], jnp.inf), axis=0)
    maxs = jnp.max(jnp.where(valid, p[:, 2], -jnp.inf), axis=0)
    cnt = jnp.maximum(sums[:, _HD - 16:_HD - 15], 1.0)
    mean = sums / cnt
    pool = jnp.concatenate(
        [mean[:, 0:100], mean[:, 112:212], mins[:, 0:100], mins[:, 112:212],
         maxs[:, 0:100], maxs[:, 112:212], sums[:, 0:100], sums[:, 112:212]],
        axis=1,
    )
    out = pool @ W1_ref[...] + b1_ref[...]
    out = out @ W2_ref[...] + b2_ref[...]
    out = out @ W3_ref[...] + b3_ref[...]
    out_ref[...] = out[:_G]


def _row_specs(widths):
    return [pl.BlockSpec((_R, w), lambda i: (i, 0)) for w in widths]


def _full_specs(shapes):
    return [
        pl.BlockSpec(s, lambda i, _n=len(s): (0,) * _n) for s in shapes
    ]


def _pad2(a, shape, r0=0, c0=0):
    return jnp.zeros(shape, jnp.float32).at[r0:r0 + a.shape[0], c0:c0 + a.shape[1]].set(a)


def kernel(x, edge_index, node_graph_index, self_k0, neigh_k0, bias0, self_k1, neigh_k1, bias1, self_kF, neigh_kF, biasF, W1, b1, W2, b2, W3, b3):
    if _EP == _E:
        col3 = edge_index[1].reshape(_NW, _CH, _C)
        row3 = edge_index[0].reshape(_NW, _CH, _C)
    else:
        col3 = jnp.concatenate(
            [edge_index[1], jnp.zeros((_EP - _E,), jnp.int32)]).reshape(_NW, _CH, _C)
        pad_rows = _N + 16 + jnp.arange(_EP - _E, dtype=jnp.int32) % (_NP - _N - 16)
        row3 = jnp.concatenate([edge_index[0], pad_rows]).reshape(_NW, _CH, _C)
    gidp = jnp.concatenate([node_graph_index, jnp.full((_NP - _N,), _G, jnp.int32)])

    nk0 = _pad2(neigh_k0, (128, 64))
    sk0 = _pad2(self_k0, (128, 64))
    w1 = [_pad2(m[a:a + 50], (64, 80)) for m in (neigh_k1, self_k1) for a in (0, 50)]
    w2 = [_pad2(m[a:a + 75], (80, 112)) for m in (neigh_kF, self_kF) for a in (0, 75)]
    bn0, bs0 = _pad2(bias0[None, :50], (1, 64)), _pad2(bias0[None, 50:], (1, 64))
    bn1, bs1 = _pad2(bias1[None, :75], (1, 80)), _pad2(bias1[None, 75:], (1, 80))
    bn2, bs2 = _pad2(biasF[None, :100], (1, 112)), _pad2(biasF[None, 100:], (1, 112))

    grid = _NP // _R

    p0, sh0 = pl.pallas_call(
        _proj_body,
        grid=grid,
        in_specs=_row_specs([128]) + _full_specs([(128, 64), (128, 64)]),
        out_specs=_row_specs([64, 64]),
        out_shape=[jax.ShapeDtypeStruct((_NP, 64), jnp.float32)] * 2,
    )(x, nk0, sk0)

    s0 = _sc_segsum(64)(p0, col3, row3)

    def combine(sp, sh, bn, bs, ws, u, uo, dpi, dpo):
        return pl.pallas_call(
            functools.partial(_combine_body, u=u, uo=uo, dpi=dpi, dpo=dpo),
            grid=grid,
            in_specs=[pl.BlockSpec((_NC, _R, dpi), lambda i: (0, i, 0))]
            + _row_specs([dpi])
            + _full_specs([(1, dpi), (1, dpi), (dpi, dpo), (dpi, dpo), (dpi, dpo), (dpi, dpo)]),
            out_specs=_row_specs([dpo, dpo]),
            out_shape=[jax.ShapeDtypeStruct((_NP, dpo), jnp.float32)] * 2,
        )(sp, sh, bn, bs, ws[0], ws[1], ws[2], ws[3])

    p1, sh1 = combine(s0, sh0, bn0, bs0, w1, 50, 75, 64, 80)
    s1 = _sc_segsum(80)(p1, col3, row3)
    p2, sh2 = combine(s1, sh1, bn1, bs1, w2, 75, 100, 80, 112)
    s2 = _sc_segsum(112)(p2, col3, row3)

    h3 = pl.pallas_call(
        _final_body,
        grid=grid,
        in_specs=[pl.BlockSpec((_NC, _R, 112), lambda i: (0, i, 0))]
        + _row_specs([112])
        + _full_specs([(1, 112), (1, 112)]),
        out_specs=pl.BlockSpec((_R, _HD), lambda i: (i, 0)),
        out_shape=jax.ShapeDtypeStruct((_NP, _HD), jnp.float32),
    )(s2, sh2, bn2, bs2)

    parts = _make_sc_pool()(h3, gidp)

    out = pl.pallas_call(
        _head_body,
        out_shape=jax.ShapeDtypeStruct((_G, 10), jnp.float32),
    )(parts, W1, b1[None], W2, b2[None], W3, b3[None])
    return out


# speedup vs baseline: 1.2020x; 1.0596x over previous
import functools

import jax
import jax.numpy as jnp
from jax import lax
from jax.experimental import pallas as pl
from jax.experimental.pallas import tpu as pltpu
from jax.experimental.pallas import tpu_sc as plsc

_N = 10000
_E = 320000
_G = 100

_NC = 2
_NS = 16
_NW = _NC * _NS
_NP = 10240
_CH = 80
_C = 125
_EP = _NW * _CH * _C

_R = 5120
_PW = _NP // _NW
_PC = 64
_GP = 104
_HD = 256
_HL = 15
_CL = 224

_sc_mesh = dict(core_axis_name="c", subcore_axis_name="s", num_cores=_NC,
                num_subcores=_NS)
_sc_params = pltpu.CompilerParams(use_tc_tiling_on_sc=False,
                                  needs_layout_passes=False)


def _make_sc_segsum(dp):
    dl = dp // 16
    zrows = 64
    rows_per_sub = _NP // _NS

    @functools.partial(
        pl.kernel,
        out_type=jax.ShapeDtypeStruct((_NC, _NP, dp), jnp.float32),
        mesh=plsc.VectorSubcoreMesh(**_sc_mesh),
        compiler_params=_sc_params,
        scratch_types=[
            pltpu.VMEM((_CH, _C), jnp.int32),
            pltpu.VMEM((_CH, _C), jnp.int32),
            pltpu.VMEM((_C, dp), jnp.float32),
            pltpu.VMEM((_C, dp), jnp.float32),
            pltpu.VMEM((zrows, dp), jnp.float32),
            pltpu.VMEM_SHARED((_NP, dp), jnp.float32),
            pltpu.SemaphoreType.DMA,
            pltpu.SemaphoreType.DMA,
        ],
    )
    def k(p_hbm, col_hbm, row_hbm, out_hbm, colv, rowv, buf0, buf1, zbuf, acc, sem0, sem1):
        cid = lax.axis_index("c")
        sid = lax.axis_index("s")
        wid = sid * _NC + cid

        def zb_body(i, carry):
            r = i // dl
            j = i - r * dl
            zbuf[r, pl.ds(j * 16, 16)] = jnp.zeros((16,), jnp.float32)
            return carry

        lax.fori_loop(0, zrows * dl, zb_body, 0)

        base = sid * rows_per_sub

        def zc_body(i, carry):
            pltpu.sync_copy(zbuf, acc.at[pl.ds(base + i * zrows, zrows)])
            return carry

        lax.fori_loop(0, rows_per_sub // zrows, zc_body, 0)

        pltpu.sync_copy(col_hbm.at[wid], colv)
        pltpu.sync_copy(row_hbm.at[wid], rowv)

        plsc.subcore_barrier()

        pltpu.async_copy(p_hbm.at[colv.at[0]], buf0, sem0)

        def ch_body(g, carry):
            k0 = 2 * g
            k1 = 2 * g + 1
            pltpu.make_async_copy(p_hbm.at[colv.at[k0]], buf0, sem0).wait()
            pltpu.async_copy(p_hbm.at[colv.at[k1]], buf1, sem1)
            pltpu.sync_copy(buf0, acc.at[rowv.at[k0]], add=True)
            pltpu.make_async_copy(p_hbm.at[colv.at[k1]], buf1, sem1).wait()

            @pl.when(g + 1 < _CH // 2)
            def _():
                pltpu.async_copy(p_hbm.at[colv.at[k1 + 1]], buf0, sem0)

            pltpu.sync_copy(buf1, acc.at[rowv.at[k1]], add=True)
            return carry

        lax.fori_loop(0, _CH // 2, ch_body, 0)

        plsc.subcore_barrier()

        pltpu.sync_copy(
            acc.at[pl.ds(base, rows_per_sub)],
            out_hbm.at[cid, pl.ds(base, rows_per_sub)],
        )

    return k


_sc_cache = {}


def _sc_segsum(dp):
    if dp not in _sc_cache:
        _sc_cache[dp] = _make_sc_segsum(dp)
    return _sc_cache[dp]


def _make_sc_pool():
    if "pool" in _sc_cache:
        return _sc_cache["pool"]
    k = functools.partial(
        pl.kernel,
        out_type=jax.ShapeDtypeStruct((_NW, 3, _GP, _HD), jnp.float32),
        mesh=plsc.VectorSubcoreMesh(**_sc_mesh),
        compiler_params=pltpu.CompilerParams(use_tc_tiling_on_sc=True,
                                             needs_layout_passes=False),
        scratch_types=[
            pltpu.VMEM((_PC, _HD), jnp.float32),
            pltpu.VMEM((_PW,), jnp.int32),
            pltpu.VMEM((_GP, _HD), jnp.float32),
            pltpu.VMEM((_GP, _HD), jnp.float32),
            pltpu.VMEM((_GP, _HD), jnp.float32),
        ],
    )(_sc_pool_body)
    _sc_cache["pool"] = k
    return k


def _sc_pool_body(h_hbm, gid_hbm, out_hbm, chunk, gidv, accs, accn, accx):
    cid = lax.axis_index("c")
    sid = lax.axis_index("s")
    wid = sid * _NC + cid
    rbase = wid * _PW

    pltpu.sync_copy(gid_hbm.at[pl.ds(rbase, _PW)], gidv)

    def zc_body(i, carry):
        accs[i, pl.ds(_CL, 16)] = jnp.zeros((16,), jnp.float32)
        return carry

    lax.fori_loop(0, _GP, zc_body, 0)

    zero = jnp.zeros((16,), jnp.float32)
    init = (jnp.int32(-1),) + tuple(zero for _ in range(3 * _HL))

    def chunk_body(c, carry):
        pltpu.sync_copy(h_hbm.at[pl.ds(rbase + c * _PC, _PC)], chunk)

        def vec_body(k, carry):
            gvec = gidv[pl.ds(c * _PC + k * 16, 16)]
            for l in range(16):
                prev = carry[0]
                ss = carry[1:1 + _HL]
                nn = carry[1 + _HL:1 + 2 * _HL]
                xx = carry[1 + 2 * _HL:]
                lane = lax.iota(jnp.int32, 16) == l
                g = jnp.max(jnp.where(lane, gvec, -1))
                is_new = g != prev

                @pl.when(jnp.logical_and(is_new, prev >= 0))
                def _():
                    for j in range(_HL):
                        accs[prev, pl.ds(j * 16, 16)] = ss[j]
                        accn[prev, pl.ds(j * 16, 16)] = nn[j]
                        accx[prev, pl.ds(j * 16, 16)] = xx[j]

                r = k * 16 + l
                vs = [chunk[r, pl.ds(j * 16, 16)] for j in range(_HL)]
                ss = tuple(jnp.where(is_new, vs[j], ss[j] + vs[j]) for j in range(_HL))
                nn = tuple(jnp.where(is_new, vs[j], jnp.minimum(nn[j], vs[j])) for j in range(_HL))
                xx = tuple(jnp.where(is_new, vs[j], jnp.maximum(xx[j], vs[j])) for j in range(_HL))
                carry = (g,) + ss + nn + xx
            return carry

        return lax.fori_loop(0, _PC // 16, vec_body, carry)

    fin = lax.fori_loop(0, _PW // _PC, chunk_body, init)
    prev = fin[0]
    for j in range(_HL):
        accs[prev, pl.ds(j * 16, 16)] = fin[1 + j]
        accn[prev, pl.ds(j * 16, 16)] = fin[1 + _HL + j]
        accx[prev, pl.ds(j * 16, 16)] = fin[1 + 2 * _HL + j]

    pltpu.sync_copy(accs, out_hbm.at[wid, 0])
    pltpu.sync_copy(accn, out_hbm.at[wid, 1])
    pltpu.sync_copy(accx, out_hbm.at[wid, 2])


def _proj_body(x_ref, nk_ref, sk_ref, pp_ref, sh_ref):
    xb = x_ref[...]
    ones = jnp.where(lax.broadcasted_iota(jnp.int32, (_R, 64), 1) == 50, 1.0, 0.0)
    pp_ref[...] = xb @ nk_ref[...] + ones
    sh_ref[...] = xb @ sk_ref[...]


def _combine_body(sp_ref, sh_ref, bn_ref, bs_ref, nkt_ref, nkb_ref, skt_ref,
                  skb_ref, pp_ref, shn_ref, *, u, uo, dpi, dpo):
    s = sp_ref[0] + sp_ref[1]
    deg = jnp.maximum(s[:, u:u + 1], 1.0)
    hm = jnp.maximum(s / deg + bn_ref[...], 0.0)
    hs = jnp.maximum(sh_ref[...] + bs_ref[...], 0.0)
    ones = jnp.where(lax.broadcasted_iota(jnp.int32, (_R, dpo), 1) == uo, 1.0, 0.0)
    pp_ref[...] = hm @ nkt_ref[...] + hs @ nkb_ref[...] + ones
    shn_ref[...] = hm @ skt_ref[...] + hs @ skb_ref[...]


def _final_body(sp_ref, sh_ref, bn_ref, bs_ref, out_ref):
    s = sp_ref[0] + sp_ref[1]
    deg = jnp.maximum(s[:, 100:101], 1.0)
    hm = jnp.maximum(s / deg + bn_ref[...], 0.0)
    hs = jnp.maximum(sh_ref[...] + bs_ref[...], 0.0)
    cnt = jnp.where(lax.broadcasted_iota(jnp.int32, (_R, 32), 1) == 0, 1.0, 0.0)
    out_ref[...] = jnp.concatenate([hm, hs, cnt], axis=1)


def _head_body(parts_ref, W1_ref, b1_ref, W2_ref, b2_ref, W3_ref, b3_ref, out_ref):
    p = parts_ref[...]
    valid = p[:, 0, :, _CL:_CL + 1] > 0.0
    sums = jnp.sum(jnp.where(valid, p[:, 0], 0.0), axis=0)
    mins = jnp.min(jnp.where(valid, p[:, 1], jnp.inf), axis=0)
    maxs = jnp.max(jnp.where(valid, p[:, 2], -jnp.inf), axis=0)
    cnt = jnp.maximum(sums[:, _CL:_CL + 1], 1.0)
    mean = sums / cnt
    pool = jnp.concatenate(
        [mean[:, 0:100], mean[:, 112:212], mins[:, 0:100], mins[:, 112:212],
         maxs[:, 0:100], maxs[:, 112:212], sums[:, 0:100], sums[:, 112:212]],
        axis=1,
    )
    out = pool @ W1_ref[...] + b1_ref[...]
    out = out @ W2_ref[...] + b2_ref[...]
    out = out @ W3_ref[...] + b3_ref[...]
    out_ref[...] = out[:_G]


def _row_specs(widths):
    return [pl.BlockSpec((_R, w), lambda i: (i, 0)) for w in widths]


def _full_specs(shapes):
    return [
        pl.BlockSpec(s, lambda i, _n=len(s): (0,) * _n) for s in shapes
    ]


def _pad2(a, shape, r0=0, c0=0):
    return jnp.zeros(shape, jnp.float32).at[r0:r0 + a.shape[0], c0:c0 + a.shape[1]].set(a)


def kernel(x, edge_index, node_graph_index, self_k0, neigh_k0, bias0, self_k1, neigh_k1, bias1, self_kF, neigh_kF, biasF, W1, b1, W2, b2, W3, b3):
    if _EP == _E:
        col3 = edge_index[1].reshape(_NW, _CH, _C)
        row3 = edge_index[0].reshape(_NW, _CH, _C)
    else:
        col3 = jnp.concatenate(
            [edge_index[1], jnp.zeros((_EP - _E,), jnp.int32)]).reshape(_NW, _CH, _C)
        pad_rows = _N + 16 + jnp.arange(_EP - _E, dtype=jnp.int32) % (_NP - _N - 16)
        row3 = jnp.concatenate([edge_index[0], pad_rows]).reshape(_NW, _CH, _C)
    gidp = jnp.concatenate([node_graph_index, jnp.full((_NP - _N,), _G, jnp.int32)])

    nk0 = _pad2(neigh_k0, (128, 64))
    sk0 = _pad2(self_k0, (128, 64))
    w1 = [_pad2(m[a:a + 50], (64, 80)) for m in (neigh_k1, self_k1) for a in (0, 50)]
    w2 = [_pad2(m[a:a + 75], (80, 112)) for m in (neigh_kF, self_kF) for a in (0, 75)]
    bn0, bs0 = _pad2(bias0[None, :50], (1, 64)), _pad2(bias0[None, 50:], (1, 64))
    bn1, bs1 = _pad2(bias1[None, :75], (1, 80)), _pad2(bias1[None, 75:], (1, 80))
    bn2, bs2 = _pad2(biasF[None, :100], (1, 112)), _pad2(biasF[None, 100:], (1, 112))

    grid = _NP // _R

    p0, sh0 = pl.pallas_call(
        _proj_body,
        grid=grid,
        in_specs=_row_specs([128]) + _full_specs([(128, 64), (128, 64)]),
        out_specs=_row_specs([64, 64]),
        out_shape=[jax.ShapeDtypeStruct((_NP, 64), jnp.float32)] * 2,
    )(x, nk0, sk0)

    s0 = _sc_segsum(64)(p0, col3, row3)

    def combine(sp, sh, bn, bs, ws, u, uo, dpi, dpo):
        return pl.pallas_call(
            functools.partial(_combine_body, u=u, uo=uo, dpi=dpi, dpo=dpo),
            grid=grid,
            in_specs=[pl.BlockSpec((_NC, _R, dpi), lambda i: (0, i, 0))]
            + _row_specs([dpi])
            + _full_specs([(1, dpi), (1, dpi), (dpi, dpo), (dpi, dpo), (dpi, dpo), (dpi, dpo)]),
            out_specs=_row_specs([dpo, dpo]),
            out_shape=[jax.ShapeDtypeStruct((_NP, dpo), jnp.float32)] * 2,
        )(sp, sh, bn, bs, ws[0], ws[1], ws[2], ws[3])

    p1, sh1 = combine(s0, sh0, bn0, bs0, w1, 50, 75, 64, 80)
    s1 = _sc_segsum(80)(p1, col3, row3)
    p2, sh2 = combine(s1, sh1, bn1, bs1, w2, 75, 100, 80, 112)
    s2 = _sc_segsum(112)(p2, col3, row3)

    h3 = pl.pallas_call(
        _final_body,
        grid=grid,
        in_specs=[pl.BlockSpec((_NC, _R, 112), lambda i: (0, i, 0))]
        + _row_specs([112])
        + _full_specs([(1, 112), (1, 112)]),
        out_specs=pl.BlockSpec((_R, _HD), lambda i: (i, 0)),
        out_shape=jax.ShapeDtypeStruct((_NP, _HD), jnp.float32),
    )(s2, sh2, bn2, bs2)

    parts = _make_sc_pool()(h3, gidp)

    out = pl.pallas_call(
        _head_body,
        out_shape=jax.ShapeDtypeStruct((_G, 10), jnp.float32),
    )(parts, W1, b1[None], W2, b2[None], W3, b3[None])
    return out
